# Initial kernel scaffold; baseline (speedup 1.0000x reference)
import jax
import jax.numpy as jnp
from jax.experimental import pallas as pl


def kernel(x, edge_index, batch, eps, W1, b1, g1, be1, W2, b2, g2, be2):
    raise NotImplementedError("write your pallas kernel here")


# speedup vs baseline: 3.8285x; 3.8285x over previous
import functools

import jax
import jax.numpy as jnp
from jax import lax
from jax.experimental import pallas as pl
from jax.experimental.pallas import tpu as pltpu
from jax.experimental.pallas import tpu_sc as plsc

N = 10000
D = 128
G = 128
L = 5
E = 320000

NC = 2
NS = 16
NW = NC * NS
CHUNK = 128
CHUNKS_PER_W = (E + NW * CHUNK - 1) // (NW * CHUNK)
E_PAD = NW * CHUNKS_PER_W * CHUNK
N_PAD = 10240
ROWS_PER_TILE = N_PAD // NS


def _sc_agg_body(h_hbm, col_hbm, row_hbm, zeros_hbm, out_hbm,
                 col_v, row_v, gbuf, sem, acc):
    c = lax.axis_index("c")
    s = lax.axis_index("s")
    wid = c * NS + s
    for z in range(ROWS_PER_TILE // 128):
        pltpu.sync_copy(zeros_hbm, acc.at[pl.ds(s * ROWS_PER_TILE + z * 128, 128)])
    pltpu.sync_copy(col_hbm.at[wid], col_v)
    pltpu.sync_copy(row_hbm.at[wid], row_v)
    plsc.subcore_barrier()

    def body(j, carry):
        pltpu.async_copy(h_hbm.at[col_v.at[j]], gbuf, sem).wait()
        pltpu.sync_copy(gbuf, acc.at[row_v.at[j]], add=True)
        return carry

    lax.fori_loop(0, CHUNKS_PER_W, body, 0)
    plsc.subcore_barrier()
    pltpu.sync_copy(acc.at[pl.ds(s * ROWS_PER_TILE, ROWS_PER_TILE)],
                    out_hbm.at[c, pl.ds(s * ROWS_PER_TILE, ROWS_PER_TILE)])


@functools.cache
def _make_sc_agg():
    return functools.partial(
        pl.kernel,
        out_type=jax.ShapeDtypeStruct((NC, N_PAD, D), jnp.float32),
        mesh=plsc.VectorSubcoreMesh(core_axis_name="c", subcore_axis_name="s"),
        scratch_types=[
            pltpu.VMEM((CHUNKS_PER_W, CHUNK), jnp.int32),
            pltpu.VMEM((CHUNKS_PER_W, CHUNK), jnp.int32),
            pltpu.VMEM((CHUNK, D), jnp.float32),
            pltpu.SemaphoreType.DMA,
            pltpu.VMEM_SHARED((N_PAD, D), jnp.float32),
        ],
    )(_sc_agg_body)


def _bn(t, gamma, beta):
    m = jnp.mean(t, axis=0, keepdims=True)
    v = jnp.mean((t - m) ** 2, axis=0, keepdims=True)
    a = v + 1e-5
    inv = lax.rsqrt(a)
    inv = inv * (1.5 - 0.5 * a * inv * inv)
    return (t - m) * inv * gamma + beta


def _mlp_body(with_gpin, part_ref, h_ref, batch_ref, scale_ref,
              w1_ref, b1_ref, g1_ref, be1_ref,
              w2_ref, b2_ref, g2_ref, be2_ref, *out_refs):
    h = h_ref[...]
    pooled = part_ref[0, :N, :] + part_ref[1, :N, :] + scale_ref[0, 0] * h
    t = jnp.dot(pooled, w1_ref[...], precision=lax.Precision.HIGHEST,
                preferred_element_type=jnp.float32) + b1_ref[...]
    t = jnp.maximum(_bn(t, g1_ref[...], be1_ref[...]), 0.0)
    t = jnp.dot(t, w2_ref[...], precision=lax.Precision.HIGHEST,
                preferred_element_type=jnp.float32) + b2_ref[...]
    h2 = jnp.maximum(_bn(t, g2_ref[...], be2_ref[...]), 0.0)
    out_refs[0][...] = h2
    onehot = (batch_ref[...] == lax.broadcasted_iota(jnp.int32, (N, G), 1)
              ).astype(jnp.float32)
    out_refs[1][...] = lax.dot_general(
        onehot, h2, (((0,), (0,)), ((), ())), precision=lax.Precision.HIGHEST,
        preferred_element_type=jnp.float32)
    if with_gpin:
        out_refs[2][...] = lax.dot_general(
            onehot, h, (((0,), (0,)), ((), ())), precision=lax.Precision.HIGHEST,
            preferred_element_type=jnp.float32)


def _mlp_call(with_gpin, *args):
    n_out = 3 if with_gpin else 2
    out_shape = [jax.ShapeDtypeStruct((N, D), jnp.float32),
                 jax.ShapeDtypeStruct((G, D), jnp.float32)]
    if with_gpin:
        out_shape.append(jax.ShapeDtypeStruct((G, D), jnp.float32))
    return pl.pallas_call(
        functools.partial(_mlp_body, with_gpin),
        out_shape=out_shape,
    )(*args)


def kernel(x, edge_index, batch, eps, W1, b1, g1, be1, W2, b2, g2, be2):
    ei = edge_index.astype(jnp.int32)
    row, col = ei[0], ei[1]
    pad = E_PAD - E
    colp = jnp.concatenate([col, jnp.zeros((pad,), jnp.int32)]
                           ).reshape(NW, CHUNKS_PER_W, CHUNK)
    rowp = jnp.concatenate([row, jnp.full((pad,), N, jnp.int32)]
                           ).reshape(NW, CHUNKS_PER_W, CHUNK)
    zeros128 = jnp.zeros((128, D), jnp.float32)
    batch2 = batch.astype(jnp.int32).reshape(N, 1)

    h = x
    gps = []
    for l in range(L):
        part = _make_sc_agg()(h, colp, rowp, zeros128)
        scale = (1.0 + eps[l]).reshape(1, 1)
        args = (part, h, batch2, scale,
                W1[l], b1[l].reshape(1, D), g1[l].reshape(1, D), be1[l].reshape(1, D),
                W2[l], b2[l].reshape(1, D), g2[l].reshape(1, D), be2[l].reshape(1, D))
        if l == 0:
            h, gp, gp0 = _mlp_call(True, *args)
            gps = [gp0, gp]
        else:
            h, gp = _mlp_call(False, *args)
            gps.append(gp)
    return (*gps, h)
